# Initial kernel scaffold; baseline (speedup 1.0000x reference)
import jax
import jax.numpy as jnp
from jax.experimental import pallas as pl


def kernel(x, position_ids, pos_table):
    raise NotImplementedError("write your pallas kernel here")


# speedup vs baseline: 2.0422x; 2.0422x over previous
import functools

import jax
import jax.numpy as jnp
from jax import lax
from jax.experimental import pallas as pl
from jax.experimental.pallas import tpu as pltpu
from jax.experimental.pallas import tpu_sc as plsc

_NUM_CORES = 2
_NUM_SUBCORES = 16
_NUM_WORKERS = _NUM_CORES * _NUM_SUBCORES


def _sc_gather(pos_table, pid):
    S, D = pos_table.shape
    rows_per_w = S // _NUM_WORKERS
    chunk = min(rows_per_w, 32)
    n_chunks = rows_per_w // chunk

    mesh = plsc.VectorSubcoreMesh(
        core_axis_name="c",
        subcore_axis_name="s",
        num_cores=_NUM_CORES,
        num_subcores=_NUM_SUBCORES,
    )

    @functools.partial(
        pl.kernel,
        out_type=jax.ShapeDtypeStruct((S, D), pos_table.dtype),
        mesh=mesh,
        scratch_types=[
            pltpu.VMEM((rows_per_w,), jnp.int32),
            pltpu.VMEM((chunk, D), pos_table.dtype),
            pltpu.VMEM((chunk, D), pos_table.dtype),
            pltpu.SemaphoreType.DMA,
            pltpu.SemaphoreType.DMA,
        ],
    )
    def gather_kernel(table_hbm, idx_hbm, out_hbm, idx_v, buf0, buf1, sem0, sem1):
        wid = lax.axis_index("s") * _NUM_CORES + lax.axis_index("c")
        base = wid * rows_per_w
        pltpu.sync_copy(idx_hbm.at[pl.ds(base, rows_per_w)], idx_v)

        bufs = (buf0, buf1)
        sems = (sem0, sem1)

        def gather_start(c, slot):
            pltpu.async_copy(
                table_hbm.at[idx_v.at[pl.ds(c * chunk, chunk)]], bufs[slot], sems[slot]
            )

        def gather_wait(slot):
            pltpu.make_async_copy(
                table_hbm.at[idx_v.at[pl.ds(0, chunk)]], bufs[slot], sems[slot]
            ).wait()

        gather_start(0, 0)
        for c in range(n_chunks):
            slot = c % 2
            if c + 1 < n_chunks:
                gather_start(c + 1, (c + 1) % 2)
            gather_wait(slot)
            pltpu.sync_copy(
                bufs[slot], out_hbm.at[pl.ds(base + c * chunk, chunk)]
            )

    return gather_kernel(pos_table, pid)


def _tc_add(x, emb):
    B, S, D = x.shape
    bs = 512

    def add_body(x_ref, e_ref, o_ref):
        o_ref[...] = x_ref[...] + e_ref[...]

    return pl.pallas_call(
        add_body,
        grid=(S // bs, B),
        in_specs=[
            pl.BlockSpec((1, bs, D), lambda s, b: (b, s, 0)),
            pl.BlockSpec((bs, D), lambda s, b: (s, 0)),
        ],
        out_specs=pl.BlockSpec((1, bs, D), lambda s, b: (b, s, 0)),
        out_shape=jax.ShapeDtypeStruct((B, S, D), x.dtype),
    )(x, emb)


@jax.jit
def kernel(x, position_ids, pos_table):
    S = x.shape[1]
    pid = position_ids.reshape(-1)[:S].astype(jnp.int32)
    emb = _sc_gather(pos_table, pid)
    return _tc_add(x, emb)


# speedup vs baseline: 2.1992x; 1.0768x over previous
import functools

import jax
import jax.numpy as jnp
from jax import lax
from jax.experimental import pallas as pl
from jax.experimental.pallas import tpu as pltpu
from jax.experimental.pallas import tpu_sc as plsc

_NUM_CORES = 2
_NUM_SUBCORES = 16
_NUM_WORKERS = _NUM_CORES * _NUM_SUBCORES


def _sc_gather(pos_table, pid):
    S, D = pos_table.shape
    rows_per_w = S // _NUM_WORKERS
    chunk = min(rows_per_w, 32)
    n_chunks = rows_per_w // chunk

    mesh = plsc.VectorSubcoreMesh(
        core_axis_name="c",
        subcore_axis_name="s",
        num_cores=_NUM_CORES,
        num_subcores=_NUM_SUBCORES,
    )

    @functools.partial(
        pl.kernel,
        out_type=jax.ShapeDtypeStruct((S, D), pos_table.dtype),
        mesh=mesh,
        scratch_types=[
            pltpu.VMEM((rows_per_w,), jnp.int32),
            pltpu.VMEM((chunk, D), pos_table.dtype),
            pltpu.VMEM((chunk, D), pos_table.dtype),
            pltpu.SemaphoreType.DMA,
            pltpu.SemaphoreType.DMA,
        ],
    )
    def gather_kernel(table_hbm, idx_hbm, out_hbm, idx_v, buf0, buf1, sem0, sem1):
        wid = lax.axis_index("s") * _NUM_CORES + lax.axis_index("c")
        base = wid * rows_per_w
        pltpu.sync_copy(idx_hbm.at[pl.ds(base, rows_per_w)], idx_v)

        bufs = (buf0, buf1)
        sems = (sem0, sem1)

        def gather_start(c, slot):
            pltpu.async_copy(
                table_hbm.at[idx_v.at[pl.ds(c * chunk, chunk)]], bufs[slot], sems[slot]
            )

        def gather_wait(slot):
            pltpu.make_async_copy(
                table_hbm.at[idx_v.at[pl.ds(0, chunk)]], bufs[slot], sems[slot]
            ).wait()

        gather_start(0, 0)
        for c in range(n_chunks):
            slot = c % 2
            if c + 1 < n_chunks:
                gather_start(c + 1, (c + 1) % 2)
            gather_wait(slot)
            pltpu.sync_copy(
                bufs[slot], out_hbm.at[pl.ds(base + c * chunk, chunk)]
            )

    return gather_kernel(pos_table, pid)


def _tc_add(x, emb):
    B, S, D = x.shape
    bs = 1024

    def add_body(x_ref, e_ref, o_ref):
        o_ref[...] = x_ref[...] + e_ref[...]

    return pl.pallas_call(
        add_body,
        grid=(S // bs, B),
        in_specs=[
            pl.BlockSpec((1, bs, D), lambda s, b: (b, s, 0)),
            pl.BlockSpec((bs, D), lambda s, b: (s, 0)),
        ],
        out_specs=pl.BlockSpec((1, bs, D), lambda s, b: (b, s, 0)),
        out_shape=jax.ShapeDtypeStruct((B, S, D), x.dtype),
    )(x, emb)


@jax.jit
def kernel(x, position_ids, pos_table):
    S = x.shape[1]
    pid = position_ids.reshape(-1)[:S].astype(jnp.int32)
    emb = _sc_gather(pos_table, pid)
    return _tc_add(x, emb)


# speedup vs baseline: 2.2637x; 1.0293x over previous
import functools

import jax
import jax.numpy as jnp
from jax import lax
from jax.experimental import pallas as pl
from jax.experimental.pallas import tpu as pltpu
from jax.experimental.pallas import tpu_sc as plsc

_NUM_CORES = 2
_NUM_SUBCORES = 16
_NUM_WORKERS = _NUM_CORES * _NUM_SUBCORES


def _sc_gather(pos_table, pid):
    S, D = pos_table.shape
    rows_per_w = S // _NUM_WORKERS
    chunk = min(rows_per_w, 32)
    n_chunks = rows_per_w // chunk

    mesh = plsc.VectorSubcoreMesh(
        core_axis_name="c",
        subcore_axis_name="s",
        num_cores=_NUM_CORES,
        num_subcores=_NUM_SUBCORES,
    )

    @functools.partial(
        pl.kernel,
        out_type=jax.ShapeDtypeStruct((S, D), pos_table.dtype),
        mesh=mesh,
        scratch_types=[
            pltpu.VMEM((rows_per_w,), jnp.int32),
            pltpu.VMEM((chunk, D), pos_table.dtype),
            pltpu.VMEM((chunk, D), pos_table.dtype),
            pltpu.SemaphoreType.DMA,
            pltpu.SemaphoreType.DMA,
        ],
    )
    def gather_kernel(table_hbm, idx_hbm, out_hbm, idx_v, buf0, buf1, sem0, sem1):
        wid = lax.axis_index("s") * _NUM_CORES + lax.axis_index("c")
        base = wid * rows_per_w
        pltpu.sync_copy(idx_hbm.at[pl.ds(base, rows_per_w)], idx_v)

        bufs = (buf0, buf1)
        sems = (sem0, sem1)

        def gather_start(c, slot):
            pltpu.async_copy(
                table_hbm.at[idx_v.at[pl.ds(c * chunk, chunk)]], bufs[slot], sems[slot]
            )

        def gather_wait(slot):
            pltpu.make_async_copy(
                table_hbm.at[idx_v.at[pl.ds(0, chunk)]], bufs[slot], sems[slot]
            ).wait()

        gather_start(0, 0)
        for c in range(n_chunks):
            slot = c % 2
            if c + 1 < n_chunks:
                gather_start(c + 1, (c + 1) % 2)
            gather_wait(slot)
            pltpu.sync_copy(
                bufs[slot], out_hbm.at[pl.ds(base + c * chunk, chunk)]
            )

    return gather_kernel(pos_table, pid)


def _tc_add(x, emb):
    B, S, D = x.shape
    bs = 2048

    def add_body(x_ref, e_ref, o_ref):
        o_ref[...] = x_ref[...] + e_ref[...]

    return pl.pallas_call(
        add_body,
        grid=(S // bs, B),
        in_specs=[
            pl.BlockSpec((1, bs, D), lambda s, b: (b, s, 0)),
            pl.BlockSpec((bs, D), lambda s, b: (s, 0)),
        ],
        out_specs=pl.BlockSpec((1, bs, D), lambda s, b: (b, s, 0)),
        out_shape=jax.ShapeDtypeStruct((B, S, D), x.dtype),
    )(x, emb)


@jax.jit
def kernel(x, position_ids, pos_table):
    S = x.shape[1]
    pid = position_ids.reshape(-1)[:S].astype(jnp.int32)
    emb = _sc_gather(pos_table, pid)
    return _tc_add(x, emb)
